# Initial kernel scaffold; baseline (speedup 1.0000x reference)
import jax
import jax.numpy as jnp
from jax.experimental import pallas as pl


def kernel(node_label_ids, adjacency_list_0, adjacency_list_1, adjacency_list_2, node_to_graph_map, num_graphs, emb_table, W_msg, W_self):
    raise NotImplementedError("write your pallas kernel here")


# speedup vs baseline: 2.3961x; 2.3961x over previous
import functools

import jax
import jax.numpy as jnp
from jax import lax
from jax.experimental import pallas as pl
from jax.experimental.pallas import tpu as pltpu
from jax.experimental.pallas import tpu_sc as plsc

N_NODES = 10000
D = 128
NUM_ET = 3
E_PER = 106667
L = 2

NW = 32
NPAD = 10240
ROWS_PER_W = NPAD // NW
ROWS_PER_TILE = NPAD // 16

CH = 128
EW = 3456
NCH = EW // CH
EPAD = NW * EW

GCH = 80
NGCH = ROWS_PER_W // GCH

_mesh = plsc.VectorSubcoreMesh(core_axis_name="c", subcore_axis_name="s")


@functools.partial(
    pl.kernel,
    out_type=jax.ShapeDtypeStruct((NPAD, D), jnp.float32),
    mesh=_mesh,
    scratch_types=[
        pltpu.VMEM((GCH,), jnp.int32),
        pltpu.VMEM((GCH, D), jnp.float32),
        pltpu.SemaphoreType.DMA,
    ],
)
def _emb_gather(table_hbm, ids_hbm, out_hbm, idx_v, rows_v, sem):
    c = lax.axis_index("c")
    s = lax.axis_index("s")
    wid = s * 2 + c
    base = wid * ROWS_PER_W
    for k in range(NGCH):
        off = base + k * GCH
        pltpu.sync_copy(ids_hbm.at[pl.ds(off, GCH)], idx_v)
        pltpu.async_copy(table_hbm.at[idx_v], rows_v, sem).wait()
        pltpu.sync_copy(rows_v, out_hbm.at[pl.ds(off, GCH)])


@functools.partial(
    pl.kernel,
    out_type=jax.ShapeDtypeStruct((2, NUM_ET, NPAD, D), jnp.float32),
    mesh=_mesh,
    scratch_types=[
        pltpu.VMEM_SHARED((NPAD, D), jnp.float32),
        pltpu.VMEM((CH,), jnp.int32),
        pltpu.VMEM((CH,), jnp.int32),
        pltpu.VMEM((CH, D), jnp.float32),
        pltpu.SemaphoreType.DMA,
    ],
)
def _seg_sums(h_hbm, src0, dst0, src1, dst1, src2, dst2, zeros_hbm, out_hbm,
              acc_sh, src_v, dst_v, rows_v, sem):
    c = lax.axis_index("c")
    s = lax.axis_index("s")
    wid = s * 2 + c
    tile_lo = s * ROWS_PER_TILE
    edge_lists = ((src0, dst0), (src1, dst1), (src2, dst2))
    for t in range(NUM_ET):
        src_hbm, dst_hbm = edge_lists[t]
        pltpu.sync_copy(zeros_hbm, acc_sh.at[pl.ds(tile_lo, ROWS_PER_TILE)])
        plsc.subcore_barrier()

        ebase = wid * EW

        def body(k, carry, src_hbm=src_hbm, dst_hbm=dst_hbm, ebase=ebase):
            off = pl.multiple_of(ebase + k * CH, CH)
            pltpu.sync_copy(src_hbm.at[pl.ds(off, CH)], src_v)
            pltpu.sync_copy(dst_hbm.at[pl.ds(off, CH)], dst_v)
            pltpu.async_copy(h_hbm.at[src_v], rows_v, sem).wait()
            pltpu.sync_copy(rows_v, acc_sh.at[dst_v], add=True)
            return carry

        lax.fori_loop(0, NCH, body, 0)
        plsc.subcore_barrier()
        pltpu.sync_copy(
            acc_sh.at[pl.ds(tile_lo, ROWS_PER_TILE)],
            out_hbm.at[c, t, pl.ds(tile_lo, ROWS_PER_TILE)],
        )


BM = 512


def _layer_body(h_ref, s_ref, wself_ref, wmsg_ref, out_ref):
    acc = jnp.dot(h_ref[...], wself_ref[...], preferred_element_type=jnp.float32)
    for t in range(NUM_ET):
        st = s_ref[0, t] + s_ref[1, t]
        acc += jnp.dot(st, wmsg_ref[t], preferred_element_type=jnp.float32)
    out_ref[...] = jnp.maximum(acc, 0.0)


def _tc_layer(h, S, wself, wmsg):
    return pl.pallas_call(
        _layer_body,
        grid=(NPAD // BM,),
        in_specs=[
            pl.BlockSpec((BM, D), lambda i: (i, 0)),
            pl.BlockSpec((2, NUM_ET, BM, D), lambda i: (0, 0, i, 0)),
            pl.BlockSpec((D, D), lambda i: (0, 0)),
            pl.BlockSpec((NUM_ET, D, D), lambda i: (0, 0, 0)),
        ],
        out_specs=pl.BlockSpec((BM, D), lambda i: (i, 0)),
        out_shape=jax.ShapeDtypeStruct((NPAD, D), jnp.float32),
    )(h, S, wself, wmsg)


def kernel(node_label_ids, adjacency_list_0, adjacency_list_1, adjacency_list_2,
           node_to_graph_map, num_graphs, emb_table, W_msg, W_self):
    ids = jnp.zeros((NPAD,), jnp.int32).at[:N_NODES].set(
        node_label_ids.astype(jnp.int32))
    srcs, dsts = [], []
    for a in (adjacency_list_0, adjacency_list_1, adjacency_list_2):
        a = a.astype(jnp.int32)
        srcs.append(jnp.zeros((EPAD,), jnp.int32).at[:E_PER].set(a[:, 0]))
        dsts.append(jnp.full((EPAD,), NPAD - 1, jnp.int32).at[:E_PER].set(a[:, 1]))
    zeros = jnp.zeros((ROWS_PER_TILE, D), jnp.float32)

    h = _emb_gather(emb_table, ids)
    for layer in range(L):
        S = _seg_sums(h, srcs[0], dsts[0], srcs[1], dsts[1], srcs[2], dsts[2],
                      zeros)
        h = _tc_layer(h, S, W_self[layer], W_msg[layer])
    return h[:N_NODES]
